# Initial kernel scaffold; baseline (speedup 1.0000x reference)
import functools

import jax
import jax.numpy as jnp
from jax import lax
from jax.experimental import pallas as pl
from jax.experimental.pallas import tpu as pltpu
from jax.experimental.pallas import tpu_sc as plsc

IOU_THRESHOLD = 0.5
MAX_DET = 300
PAD_DET = 304
BATCH = 8
NUM_BOXES = 20000
L = 16
NBLK = PAD_DET // L
INT32_MAX = 2147483647


def _nms_body(sidx_hbm, x1_hbm, y1_hbm, x2_hbm, y2_hbm, scores_hbm, cls_hbm,
              keep_out, score_out, bx1_out, by1_out, bx2_out, by2_out,
              cls_out, cnt_out,
              sidx_v, x1_v, y1_v, x2_v, y2_v, cls_v,
              kx1, ky1, kx2, ky2, karea,
              st_keep, st_safe, st_score, st_cls, st_cnt):
    c = lax.axis_index("c")
    s = lax.axis_index("s")
    wid = s * 2 + c

    @pl.when(wid < BATCH)
    def _():
        b = wid
        pltpu.sync_copy(sidx_hbm.at[b], sidx_v)
        pltpu.sync_copy(x1_hbm.at[b], x1_v)
        pltpu.sync_copy(y1_hbm.at[b], y1_v)
        pltpu.sync_copy(x2_hbm.at[b], x2_v)
        pltpu.sync_copy(y2_hbm.at[b], y2_v)

        zf = jnp.zeros((L,), jnp.float32)
        neg1 = jnp.full((L,), -1, jnp.int32)
        zi = jnp.zeros((L,), jnp.int32)
        for k in range(NBLK):
            sl = pl.ds(k * L, L)
            kx1[sl] = zf
            ky1[sl] = zf
            kx2[sl] = zf
            ky2[sl] = zf
            karea[sl] = zf
            st_keep[sl] = neg1
            st_safe[sl] = zi

        lane0 = lax.iota(jnp.int32, L) == 0

        def scan_body(state):
            pos, cnt = state
            idx16 = sidx_v[pl.ds(pos, L)]
            cx1 = plsc.load_gather(x1_v, [idx16])
            cy1 = plsc.load_gather(y1_v, [idx16])
            cx2 = plsc.load_gather(x2_v, [idx16])
            cy2 = plsc.load_gather(y2_v, [idx16])
            careas = (cx2 - cx1) * (cy2 - cy1)

            for j in range(L):
                ax1 = jnp.full((L,), cx1[j])
                ay1 = jnp.full((L,), cy1[j])
                ax2 = jnp.full((L,), cx2[j])
                ay2 = jnp.full((L,), cy2[j])
                aar = jnp.full((L,), careas[j])
                nblk = (cnt + (L - 1)) // L

                def iou_blk(k, acc):
                    sl = pl.ds(k * L, L)
                    w = jnp.maximum(
                        jnp.minimum(kx2[sl], ax2) - jnp.maximum(kx1[sl], ax1),
                        0.0)
                    h = jnp.maximum(
                        jnp.minimum(ky2[sl], ay2) - jnp.maximum(ky1[sl], ay1),
                        0.0)
                    inter = w * h
                    denom = karea[sl] + aar - inter
                    return jnp.maximum(acc, inter - IOU_THRESHOLD * denom)

                metric = lax.fori_loop(0, nblk, iou_blk,
                                       jnp.full((L,), -1.0, jnp.float32))
                mmax = lax.reduce_max_p.bind(metric, axes=(0,))
                keep_j = (cnt < MAX_DET) & (mmax <= 0.0)

                @pl.when(keep_j)
                def _():
                    idxv = jnp.full((L,), cnt, jnp.int32)
                    plsc.store_scatter(kx1, [idxv], ax1, lane0)
                    plsc.store_scatter(ky1, [idxv], ay1, lane0)
                    plsc.store_scatter(kx2, [idxv], ax2, lane0)
                    plsc.store_scatter(ky2, [idxv], ay2, lane0)
                    plsc.store_scatter(karea, [idxv], aar, lane0)
                    oidx = jnp.full((L,), idx16[j], jnp.int32)
                    plsc.store_scatter(st_keep, [idxv], oidx, lane0)
                    plsc.store_scatter(st_safe, [idxv], oidx, lane0)

                cnt = cnt + keep_j.astype(jnp.int32)
            return pos + L, cnt

        def scan_cond(state):
            pos, cnt = state
            return (pos < NUM_BOXES) & (cnt < MAX_DET)

        _, cnt = lax.while_loop(scan_cond, scan_body,
                                (jnp.int32(0), jnp.int32(0)))

        st_cnt[...] = jnp.full((L,), cnt, jnp.int32)

        pltpu.sync_copy(scores_hbm.at[b], x1_v)
        pltpu.sync_copy(cls_hbm.at[b], cls_v)
        for k in range(NBLK):
            sl = pl.ds(k * L, L)
            kidx = st_safe[sl]
            raw = st_keep[sl]
            valid = raw >= 0
            sc = plsc.load_gather(x1_v, [kidx])
            cl = plsc.load_gather(cls_v, [kidx])
            st_score[sl] = jnp.where(valid, sc, 0.0)
            st_cls[sl] = jnp.where(valid, cl, INT32_MAX)

        pltpu.sync_copy(st_keep, keep_out.at[b])
        pltpu.sync_copy(st_score, score_out.at[b])
        pltpu.sync_copy(kx1, bx1_out.at[b])
        pltpu.sync_copy(ky1, by1_out.at[b])
        pltpu.sync_copy(kx2, bx2_out.at[b])
        pltpu.sync_copy(ky2, by2_out.at[b])
        pltpu.sync_copy(st_cls, cls_out.at[b])
        pltpu.sync_copy(st_cnt, cnt_out.at[b])


_sc_nms = functools.partial(
    pl.kernel,
    out_type=(
        jax.ShapeDtypeStruct((BATCH, PAD_DET), jnp.int32),
        jax.ShapeDtypeStruct((BATCH, PAD_DET), jnp.float32),
        jax.ShapeDtypeStruct((BATCH, PAD_DET), jnp.float32),
        jax.ShapeDtypeStruct((BATCH, PAD_DET), jnp.float32),
        jax.ShapeDtypeStruct((BATCH, PAD_DET), jnp.float32),
        jax.ShapeDtypeStruct((BATCH, PAD_DET), jnp.float32),
        jax.ShapeDtypeStruct((BATCH, PAD_DET), jnp.int32),
        jax.ShapeDtypeStruct((BATCH, L), jnp.int32),
    ),
    mesh=plsc.VectorSubcoreMesh(core_axis_name="c", subcore_axis_name="s"),
    scratch_types=[
        pltpu.VMEM((NUM_BOXES,), jnp.int32),
        pltpu.VMEM((NUM_BOXES,), jnp.float32),
        pltpu.VMEM((NUM_BOXES,), jnp.float32),
        pltpu.VMEM((NUM_BOXES,), jnp.float32),
        pltpu.VMEM((NUM_BOXES,), jnp.float32),
        pltpu.VMEM((NUM_BOXES,), jnp.int32),
        pltpu.VMEM((PAD_DET,), jnp.float32),
        pltpu.VMEM((PAD_DET,), jnp.float32),
        pltpu.VMEM((PAD_DET,), jnp.float32),
        pltpu.VMEM((PAD_DET,), jnp.float32),
        pltpu.VMEM((PAD_DET,), jnp.float32),
        pltpu.VMEM((PAD_DET,), jnp.int32),
        pltpu.VMEM((PAD_DET,), jnp.int32),
        pltpu.VMEM((PAD_DET,), jnp.float32),
        pltpu.VMEM((PAD_DET,), jnp.int32),
        pltpu.VMEM((L,), jnp.int32),
    ],
)(_nms_body)


def kernel(scores, boxes, classes):
    B, N = scores.shape
    iota = lax.broadcasted_iota(jnp.int32, (B, N), 1)
    _, sidx = lax.sort_key_val(-scores, iota, dimension=1, is_stable=True)
    x1 = boxes[..., 0]
    y1 = boxes[..., 1]
    x2 = boxes[..., 2]
    y2 = boxes[..., 3]
    keep, osc, ox1, oy1, ox2, oy2, ocl, ocnt = _sc_nms(
        sidx, x1, y1, x2, y2, scores, classes.astype(jnp.int32))
    out_boxes = jnp.stack([ox1, oy1, ox2, oy2], axis=-1)[:, :MAX_DET, :]
    return (
        keep[:, :MAX_DET],
        osc[:, :MAX_DET],
        out_boxes,
        ocl[:, :MAX_DET],
        ocnt[:, 0],
    )


# speedup vs baseline: 5.6086x; 5.6086x over previous
import functools

import jax
import jax.numpy as jnp
from jax import lax
from jax.experimental import pallas as pl
from jax.experimental.pallas import tpu as pltpu
from jax.experimental.pallas import tpu_sc as plsc

IOU_THRESHOLD = 0.5
MAX_DET = 300
PAD_DET = 304
BATCH = 8
NUM_BOXES = 20000
L = 16
NBLK = PAD_DET // L
INT32_MAX = 2147483647


def _nms_body(sidx_hbm, x1_hbm, y1_hbm, x2_hbm, y2_hbm, scores_hbm, cls_hbm,
              keep_out, score_out, bx1_out, by1_out, bx2_out, by2_out,
              cls_out, cnt_out,
              sidx_v, x1_v, y1_v, x2_v, y2_v, cls_v,
              kx1, ky1, kx2, ky2, karea,
              st_keep, st_safe, st_score, st_cls, st_cnt, cnt_smem):
    c = lax.axis_index("c")
    s = lax.axis_index("s")
    wid = s * 2 + c

    @pl.when(wid < BATCH)
    def _():
        b = wid
        pltpu.sync_copy(sidx_hbm.at[b], sidx_v)
        pltpu.sync_copy(x1_hbm.at[b], x1_v)
        pltpu.sync_copy(y1_hbm.at[b], y1_v)
        pltpu.sync_copy(x2_hbm.at[b], x2_v)
        pltpu.sync_copy(y2_hbm.at[b], y2_v)

        zf = jnp.zeros((L,), jnp.float32)
        neg1 = jnp.full((L,), -1, jnp.int32)
        zi = jnp.zeros((L,), jnp.int32)
        for k in range(NBLK):
            sl = pl.ds(k * L, L)
            kx1[sl] = zf
            ky1[sl] = zf
            kx2[sl] = zf
            ky2[sl] = zf
            karea[sl] = zf
            st_keep[sl] = neg1
            st_safe[sl] = zi

        lane0 = lax.iota(jnp.int32, L) == 0
        cnt_smem[0] = jnp.int32(0)

        def scan_blk(blk, carry):
            @pl.when(cnt_smem[0] < MAX_DET)
            def _():
                pos = blk * L
                idx16 = sidx_v[pl.ds(pos, L)]
                cx1 = plsc.load_gather(x1_v, [idx16])
                cy1 = plsc.load_gather(y1_v, [idx16])
                cx2 = plsc.load_gather(x2_v, [idx16])
                cy2 = plsc.load_gather(y2_v, [idx16])
                careas = (cx2 - cx1) * (cy2 - cy1)

                for j in range(L):
                    cnt = cnt_smem[0]
                    ax1 = jnp.full((L,), cx1[j])
                    ay1 = jnp.full((L,), cy1[j])
                    ax2 = jnp.full((L,), cx2[j])
                    ay2 = jnp.full((L,), cy2[j])
                    aar = jnp.full((L,), careas[j])

                    def iou_blk(k, acc):
                        sl = pl.ds(k * L, L)
                        w = jnp.maximum(
                            jnp.minimum(kx2[sl], ax2)
                            - jnp.maximum(kx1[sl], ax1), 0.0)
                        h = jnp.maximum(
                            jnp.minimum(ky2[sl], ay2)
                            - jnp.maximum(ky1[sl], ay1), 0.0)
                        inter = w * h
                        denom = karea[sl] + aar - inter
                        return jnp.maximum(acc, inter - IOU_THRESHOLD * denom)

                    metric = lax.fori_loop(0, NBLK, iou_blk,
                                           jnp.full((L,), -1.0, jnp.float32))
                    mmax = lax.reduce_max_p.bind(metric, axes=(0,))
                    keep_j = (cnt < MAX_DET) & (mmax <= 0.0)
                    m = lane0 & jnp.full((L,), keep_j)
                    idxv = jnp.full((L,), cnt, jnp.int32)
                    plsc.store_scatter(kx1, [idxv], ax1, mask=m)
                    plsc.store_scatter(ky1, [idxv], ay1, mask=m)
                    plsc.store_scatter(kx2, [idxv], ax2, mask=m)
                    plsc.store_scatter(ky2, [idxv], ay2, mask=m)
                    plsc.store_scatter(karea, [idxv], aar, mask=m)
                    oidx = jnp.full((L,), idx16[j], jnp.int32)
                    plsc.store_scatter(st_keep, [idxv], oidx, mask=m)
                    plsc.store_scatter(st_safe, [idxv], oidx, mask=m)
                    cnt_smem[0] = cnt + keep_j.astype(jnp.int32)

            return carry

        lax.fori_loop(0, NUM_BOXES // L, scan_blk, jnp.int32(0))
        cnt = cnt_smem[0]
        st_cnt[...] = jnp.full((L,), cnt, jnp.int32)

        pltpu.sync_copy(scores_hbm.at[b], x1_v)
        pltpu.sync_copy(cls_hbm.at[b], cls_v)
        for k in range(NBLK):
            sl = pl.ds(k * L, L)
            kidx = st_safe[sl]
            raw = st_keep[sl]
            valid = raw >= 0
            sc = plsc.load_gather(x1_v, [kidx])
            cl = plsc.load_gather(cls_v, [kidx])
            st_score[sl] = jnp.where(valid, sc, 0.0)
            st_cls[sl] = jnp.where(valid, cl, INT32_MAX)

        pltpu.sync_copy(st_keep, keep_out.at[b])
        pltpu.sync_copy(st_score, score_out.at[b])
        pltpu.sync_copy(kx1, bx1_out.at[b])
        pltpu.sync_copy(ky1, by1_out.at[b])
        pltpu.sync_copy(kx2, bx2_out.at[b])
        pltpu.sync_copy(ky2, by2_out.at[b])
        pltpu.sync_copy(st_cls, cls_out.at[b])
        pltpu.sync_copy(st_cnt, cnt_out.at[b])


_sc_nms = functools.partial(
    pl.kernel,
    out_type=(
        jax.ShapeDtypeStruct((BATCH, PAD_DET), jnp.int32),
        jax.ShapeDtypeStruct((BATCH, PAD_DET), jnp.float32),
        jax.ShapeDtypeStruct((BATCH, PAD_DET), jnp.float32),
        jax.ShapeDtypeStruct((BATCH, PAD_DET), jnp.float32),
        jax.ShapeDtypeStruct((BATCH, PAD_DET), jnp.float32),
        jax.ShapeDtypeStruct((BATCH, PAD_DET), jnp.float32),
        jax.ShapeDtypeStruct((BATCH, PAD_DET), jnp.int32),
        jax.ShapeDtypeStruct((BATCH, L), jnp.int32),
    ),
    mesh=plsc.VectorSubcoreMesh(core_axis_name="c", subcore_axis_name="s"),
    compiler_params=pltpu.CompilerParams(needs_layout_passes=False),
    scratch_types=[
        pltpu.VMEM((NUM_BOXES,), jnp.int32),
        pltpu.VMEM((NUM_BOXES,), jnp.float32),
        pltpu.VMEM((NUM_BOXES,), jnp.float32),
        pltpu.VMEM((NUM_BOXES,), jnp.float32),
        pltpu.VMEM((NUM_BOXES,), jnp.float32),
        pltpu.VMEM((NUM_BOXES,), jnp.int32),
        pltpu.VMEM((PAD_DET,), jnp.float32),
        pltpu.VMEM((PAD_DET,), jnp.float32),
        pltpu.VMEM((PAD_DET,), jnp.float32),
        pltpu.VMEM((PAD_DET,), jnp.float32),
        pltpu.VMEM((PAD_DET,), jnp.float32),
        pltpu.VMEM((PAD_DET,), jnp.int32),
        pltpu.VMEM((PAD_DET,), jnp.int32),
        pltpu.VMEM((PAD_DET,), jnp.float32),
        pltpu.VMEM((PAD_DET,), jnp.int32),
        pltpu.VMEM((L,), jnp.int32),
        pltpu.SMEM((1,), jnp.int32),
    ],
)(_nms_body)


def kernel(scores, boxes, classes):
    B, N = scores.shape
    iota = lax.broadcasted_iota(jnp.int32, (B, N), 1)
    _, sidx = lax.sort_key_val(-scores, iota, dimension=1, is_stable=True)
    x1 = boxes[..., 0]
    y1 = boxes[..., 1]
    x2 = boxes[..., 2]
    y2 = boxes[..., 3]
    keep, osc, ox1, oy1, ox2, oy2, ocl, ocnt = _sc_nms(
        sidx, x1, y1, x2, y2, scores, classes.astype(jnp.int32))
    out_boxes = jnp.stack([ox1, oy1, ox2, oy2], axis=-1)[:, :MAX_DET, :]
    return (
        keep[:, :MAX_DET],
        osc[:, :MAX_DET],
        out_boxes,
        ocl[:, :MAX_DET],
        ocnt[:, 0],
    )


# speedup vs baseline: 6.2554x; 1.1153x over previous
import functools

import jax
import jax.numpy as jnp
from jax import lax
from jax.experimental import pallas as pl
from jax.experimental.pallas import tpu as pltpu
from jax.experimental.pallas import tpu_sc as plsc

IOU_THRESHOLD = 0.5
MAX_DET = 300
PAD_DET = 304
BATCH = 8
NUM_BOXES = 20000
L = 16
NBLK = PAD_DET // L
INT32_MAX = 2147483647


def _nms_body(sidx_hbm, x1_hbm, y1_hbm, x2_hbm, y2_hbm, scores_hbm, cls_hbm,
              keep_out, score_out, bx1_out, by1_out, bx2_out, by2_out,
              cls_out, cnt_out,
              sidx_v, x1_v, y1_v, x2_v, y2_v, cls_v,
              kx1, ky1, kx2, ky2, karea,
              st_keep, st_safe, st_score, st_cls, st_cnt, cnt_smem):
    c = lax.axis_index("c")
    s = lax.axis_index("s")
    wid = s * 2 + c

    @pl.when(wid < BATCH)
    def _():
        b = wid
        pltpu.sync_copy(sidx_hbm.at[b], sidx_v)
        pltpu.sync_copy(x1_hbm.at[b], x1_v)
        pltpu.sync_copy(y1_hbm.at[b], y1_v)
        pltpu.sync_copy(x2_hbm.at[b], x2_v)
        pltpu.sync_copy(y2_hbm.at[b], y2_v)

        zf = jnp.zeros((L,), jnp.float32)
        neg1 = jnp.full((L,), -1, jnp.int32)
        zi = jnp.zeros((L,), jnp.int32)
        for k in range(NBLK):
            sl = pl.ds(k * L, L)
            kx1[sl] = zf
            ky1[sl] = zf
            kx2[sl] = zf
            ky2[sl] = zf
            karea[sl] = zf
            st_keep[sl] = neg1
            st_safe[sl] = zi

        lane0 = lax.iota(jnp.int32, L) == 0
        cnt_smem[0] = jnp.int32(0)

        def scan_blk(blk, carry):
            @pl.when(cnt_smem[0] < MAX_DET)
            def _():
                pos = blk * L
                idx16 = sidx_v[pl.ds(pos, L)]
                cx1 = plsc.load_gather(x1_v, [idx16])
                cy1 = plsc.load_gather(y1_v, [idx16])
                cx2 = plsc.load_gather(x2_v, [idx16])
                cy2 = plsc.load_gather(y2_v, [idx16])
                careas = (cx2 - cx1) * (cy2 - cy1)

                for j in range(L):
                    cnt = cnt_smem[0]
                    ax1 = jnp.full((L,), cx1[j])
                    ay1 = jnp.full((L,), cy1[j])
                    ax2 = jnp.full((L,), cx2[j])
                    ay2 = jnp.full((L,), cy2[j])
                    aar = jnp.full((L,), careas[j])

                    def iou_blk(k, acc):
                        sl = pl.ds(k * L, L)
                        w = jnp.maximum(
                            jnp.minimum(kx2[sl], ax2)
                            - jnp.maximum(kx1[sl], ax1), 0.0)
                        h = jnp.maximum(
                            jnp.minimum(ky2[sl], ay2)
                            - jnp.maximum(ky1[sl], ay1), 0.0)
                        inter = w * h
                        denom = karea[sl] + aar - inter
                        return jnp.maximum(acc, inter - IOU_THRESHOLD * denom)

                    metric = lax.fori_loop(0, NBLK, iou_blk,
                                           jnp.full((L,), -1.0, jnp.float32))
                    mmax = lax.reduce_max_p.bind(metric, axes=(0,))
                    keep_j = (cnt < MAX_DET) & (mmax <= 0.0)
                    m = lane0 & jnp.full((L,), keep_j)
                    idxv = jnp.full((L,), cnt, jnp.int32)
                    plsc.store_scatter(kx1, [idxv], ax1, mask=m)
                    plsc.store_scatter(ky1, [idxv], ay1, mask=m)
                    plsc.store_scatter(kx2, [idxv], ax2, mask=m)
                    plsc.store_scatter(ky2, [idxv], ay2, mask=m)
                    plsc.store_scatter(karea, [idxv], aar, mask=m)
                    oidx = jnp.full((L,), idx16[j], jnp.int32)
                    plsc.store_scatter(st_keep, [idxv], oidx, mask=m)
                    plsc.store_scatter(st_safe, [idxv], oidx, mask=m)
                    cnt_smem[0] = cnt + keep_j.astype(jnp.int32)

            return carry

        lax.fori_loop(0, NUM_BOXES // L, scan_blk, jnp.int32(0))
        cnt = cnt_smem[0]
        st_cnt[...] = jnp.full((L,), cnt, jnp.int32)

        pltpu.sync_copy(scores_hbm.at[b], x1_v)
        pltpu.sync_copy(cls_hbm.at[b], cls_v)
        for k in range(NBLK):
            sl = pl.ds(k * L, L)
            kidx = st_safe[sl]
            raw = st_keep[sl]
            valid = raw >= 0
            sc = plsc.load_gather(x1_v, [kidx])
            cl = plsc.load_gather(cls_v, [kidx])
            st_score[sl] = jnp.where(valid, sc, 0.0)
            st_cls[sl] = jnp.where(valid, cl, INT32_MAX)

        pltpu.sync_copy(st_keep, keep_out.at[b])
        pltpu.sync_copy(st_score, score_out.at[b])
        pltpu.sync_copy(kx1, bx1_out.at[b])
        pltpu.sync_copy(ky1, by1_out.at[b])
        pltpu.sync_copy(kx2, bx2_out.at[b])
        pltpu.sync_copy(ky2, by2_out.at[b])
        pltpu.sync_copy(st_cls, cls_out.at[b])
        pltpu.sync_copy(st_cnt, cnt_out.at[b])


_sc_nms = functools.partial(
    pl.kernel,
    out_type=(
        jax.ShapeDtypeStruct((BATCH, PAD_DET), jnp.int32),
        jax.ShapeDtypeStruct((BATCH, PAD_DET), jnp.float32),
        jax.ShapeDtypeStruct((BATCH, PAD_DET), jnp.float32),
        jax.ShapeDtypeStruct((BATCH, PAD_DET), jnp.float32),
        jax.ShapeDtypeStruct((BATCH, PAD_DET), jnp.float32),
        jax.ShapeDtypeStruct((BATCH, PAD_DET), jnp.float32),
        jax.ShapeDtypeStruct((BATCH, PAD_DET), jnp.int32),
        jax.ShapeDtypeStruct((BATCH, L), jnp.int32),
    ),
    mesh=plsc.VectorSubcoreMesh(core_axis_name="c", subcore_axis_name="s"),
    compiler_params=pltpu.CompilerParams(needs_layout_passes=False),
    scratch_types=[
        pltpu.VMEM((NUM_BOXES,), jnp.int32),
        pltpu.VMEM((NUM_BOXES,), jnp.float32),
        pltpu.VMEM((NUM_BOXES,), jnp.float32),
        pltpu.VMEM((NUM_BOXES,), jnp.float32),
        pltpu.VMEM((NUM_BOXES,), jnp.float32),
        pltpu.VMEM((NUM_BOXES,), jnp.int32),
        pltpu.VMEM((PAD_DET,), jnp.float32),
        pltpu.VMEM((PAD_DET,), jnp.float32),
        pltpu.VMEM((PAD_DET,), jnp.float32),
        pltpu.VMEM((PAD_DET,), jnp.float32),
        pltpu.VMEM((PAD_DET,), jnp.float32),
        pltpu.VMEM((PAD_DET,), jnp.int32),
        pltpu.VMEM((PAD_DET,), jnp.int32),
        pltpu.VMEM((PAD_DET,), jnp.float32),
        pltpu.VMEM((PAD_DET,), jnp.int32),
        pltpu.VMEM((L,), jnp.int32),
        pltpu.SMEM((1,), jnp.int32),
    ],
)(_nms_body)


def kernel(scores, boxes, classes):
    B, N = scores.shape
    _, sidx = lax.top_k(scores, N)
    sidx = sidx.astype(jnp.int32)
    x1 = boxes[..., 0]
    y1 = boxes[..., 1]
    x2 = boxes[..., 2]
    y2 = boxes[..., 3]
    keep, osc, ox1, oy1, ox2, oy2, ocl, ocnt = _sc_nms(
        sidx, x1, y1, x2, y2, scores, classes.astype(jnp.int32))
    out_boxes = jnp.stack([ox1, oy1, ox2, oy2], axis=-1)[:, :MAX_DET, :]
    return (
        keep[:, :MAX_DET],
        osc[:, :MAX_DET],
        out_boxes,
        ocl[:, :MAX_DET],
        ocnt[:, 0],
    )
